# baseline (speedup 1.0000x reference)
import functools

import jax
import jax.numpy as jnp
from jax import lax
from jax.experimental import pallas as pl
from jax.experimental.pallas import tpu as pltpu, tpu_sc as plsc

_info = plsc.get_sparse_core_info()
_NC, _NS = _info.num_cores, _info.num_subcores
_NW = _NC * _NS


def _worker_id():
    return lax.axis_index("s") * _NC + lax.axis_index("c")


@functools.lru_cache(maxsize=None)
def _make_table_transpose(vocab: int, hidden: int):
    assert hidden == 32 and vocab % 4 == 0
    n_full = vocab // 128
    tail = vocab - n_full * 128
    per_w = n_full // _NW
    n_extra = n_full - per_w * _NW
    assert per_w % 2 == 0 and tail % 4 == 0
    mesh = plsc.VectorSubcoreMesh(core_axis_name="c", subcore_axis_name="s")

    @functools.partial(
        pl.kernel,
        mesh=mesh,
        out_type=jax.ShapeDtypeStruct((vocab // 4, 128), jnp.float32),
        scratch_types=[
            pltpu.VMEM((2, 32, 128), jnp.float32),
            pltpu.VMEM((2, 32, 128), jnp.float32),
            pltpu.SemaphoreType.DMA,
            pltpu.SemaphoreType.DMA,
            pltpu.SemaphoreType.DMA,
            pltpu.SemaphoreType.DMA,
        ],
        compiler_params=pltpu.CompilerParams(use_tc_tiling_on_sc=True, needs_layout_passes=False),
    )
    def table_transpose(table_t, tl, in_v, out_v, is0, is1, os0, os1):
        w = _worker_id()
        start = w * per_w
        isems = (is0, is1)
        osems = (os0, os1)
        iota = lax.iota(jnp.int32, 16)

        def in_copy(i, b, sem):
            c = start + i
            return pltpu.make_async_copy(
                table_t.at[:, pl.ds(c * 128, 128)], in_v.at[b], sem)

        def out_copy(i, b, sem):
            c = start + i
            return pltpu.make_async_copy(
                out_v.at[b], tl.at[pl.ds(c * 32, 32)], sem)

        def transpose_block(in_ref, out_ref, ncols):
            @plsc.parallel_loop(0, ncols // 4, 1, unroll=4)
            def _(pj):
                for qq in range(4):
                    col = jnp.zeros((16,), jnp.int32) + (pj * 4 + qq)
                    v0 = plsc.load_gather(in_ref, [iota, col])
                    v1 = plsc.load_gather(in_ref, [iota + 16, col])
                    out_ref[pj, pl.ds(qq * 32, 16)] = v0
                    out_ref[pj, pl.ds(qq * 32 + 16, 16)] = v1

        in_copy(0, 0, is0).start()

        def pair_body(p, carry):
            for b in (0, 1):
                i = 2 * p + b

                @pl.when(i < per_w - 1)
                def _():
                    in_copy(i + 1, 1 - b, isems[1 - b]).start()

                in_copy(i, b, isems[b]).wait()

                @pl.when(i >= 2)
                def _():
                    out_copy(i - 2, b, osems[b]).wait()

                transpose_block(in_v.at[b], out_v.at[b], 128)
                out_copy(i, b, osems[b]).start()
            return carry

        lax.fori_loop(0, per_w // 2, pair_body, 0)
        out_copy(per_w - 2, 0, os0).wait()
        out_copy(per_w - 1, 1, os1).wait()

        @pl.when(w < n_extra)
        def _():
            c = per_w * _NW + w
            pltpu.sync_copy(table_t.at[:, pl.ds(c * 128, 128)], in_v.at[0])
            transpose_block(in_v.at[0], out_v.at[0], 128)
            pltpu.sync_copy(out_v.at[0], tl.at[pl.ds(c * 32, 32)])


    return table_transpose


@functools.lru_cache(maxsize=None)
def _make_gather(batch: int, seq: int, vocab: int, hidden: int):
    assert hidden == 32 and batch % 2048 == 0 and seq % 8 == 0
    t_per_w = seq // 8
    n_chunks = t_per_w * 2
    assert n_chunks % 2 == 0
    mesh = plsc.VectorSubcoreMesh(core_axis_name="c", subcore_axis_name="s")

    @functools.partial(
        pl.kernel,
        mesh=mesh,
        out_type=jax.ShapeDtypeStruct((seq, 4, batch // 128, 8, 128),
                                      jnp.float32),
        scratch_types=[
            pltpu.VMEM((2, 512), jnp.int32),
            pltpu.VMEM((2, 512, 32), jnp.float32),
            pltpu.VMEM((4, 4, 8, 128), jnp.float32),
            pltpu.SemaphoreType.DMA,
            pltpu.SemaphoreType.DMA,
            pltpu.SemaphoreType.DMA,
        ],
        compiler_params=pltpu.CompilerParams(use_tc_tiling_on_sc=False, needs_layout_passes=False),
    )
    def gather_kernel(ids_t, table_lin, out, idx_v, rows_v, out_v,
                      gs0, gs1, osem):
        w = _worker_id()
        tb = w >> 2
        bb = w & 3
        gsems = (gs0, gs1)
        iota = lax.iota(jnp.int32, 16)

        def t_of(c):
            return tb * t_per_w + (c >> 1)

        def idx_load(c, b):
            b0 = bb * 1024 + (c & 1) * 512
            pltpu.sync_copy(ids_t.at[t_of(c), pl.ds(b0, 512)], idx_v.at[b])

        def gather(b, sem):
            return pltpu.make_async_copy(
                table_lin.at[idx_v.at[b]], rows_v.at[b], sem)

        def out_dma(c, sem):
            wb = bb * 8 + (c & 1) * 4
            return pltpu.make_async_copy(
                out_v, out.at[t_of(c), :, pl.ds(wb, 4)], sem)

        def transpose_chunk(b):
            rows = rows_v.at[b]

            @plsc.parallel_loop(0, 128, 1, unroll=4)
            def _(k):
                hb = k >> 5
                wl = (k >> 3) & 3
                s8 = k & 7
                col = jnp.zeros((16,), jnp.int32) + (hb * 8 + s8)
                for l0 in range(8):
                    row = wl * 128 + l0 * 16 + iota
                    vec = plsc.load_gather(rows, [row, col])
                    out_v[hb, wl, s8, pl.ds(l0 * 16, 16)] = vec

        idx_load(0, 0)
        gather(0, gs0).start()

        def pair_body(p, carry):
            for b in (0, 1):
                c = 2 * p + b

                @pl.when(c < n_chunks - 1)
                def _():
                    idx_load(c + 1, 1 - b)
                    gather(1 - b, gsems[1 - b]).start()

                gather(b, gsems[b]).wait()

                @pl.when(c >= 1)
                def _():
                    out_dma(c - 1, osem).wait()

                transpose_chunk(b)
                out_dma(c, osem).start()
            return carry

        lax.fori_loop(0, n_chunks // 2, pair_body, 0)
        out_dma(n_chunks - 1, osem).wait()

    return gather_kernel


def kernel(input_ids, embedding_weight):
    b, t = input_ids.shape
    vocab, hidden = embedding_weight.shape
    table_t = embedding_weight.T
    tl = _make_table_transpose(vocab, hidden)(table_t)
    n_main = (vocab // 128) * 128
    if n_main < vocab:
        tail = embedding_weight[n_main:].reshape(-1, 128)
        tl = lax.dynamic_update_slice(tl, tail, (n_main // 4, 0))
    table_lin = tl.reshape(vocab, hidden)
    ids_t = input_ids.T.astype(jnp.int32)
    out_lin = _make_gather(b, t, vocab, hidden)(ids_t, table_lin)
    return out_lin.transpose(2, 4, 0, 1, 3).reshape(b, t, hidden)


# speedup vs baseline: 2.4546x; 2.4546x over previous
import functools

import jax
import jax.numpy as jnp
from jax import lax
from jax.experimental import pallas as pl
from jax.experimental.pallas import tpu as pltpu, tpu_sc as plsc

_info = plsc.get_sparse_core_info()
_NC, _NS = _info.num_cores, _info.num_subcores
_NW = _NC * _NS


def _worker_id():
    return lax.axis_index("s") * _NC + lax.axis_index("c")


@functools.lru_cache(maxsize=None)
def _make_table_transpose(vocab: int, hidden: int):
    assert hidden == 32 and vocab % 4 == 0
    n_full = vocab // 128
    per_w = n_full // _NW
    n_extra = n_full - per_w * _NW
    assert per_w % 2 == 0
    mesh = plsc.VectorSubcoreMesh(core_axis_name="c", subcore_axis_name="s")

    @functools.partial(
        pl.kernel,
        mesh=mesh,
        out_type=jax.ShapeDtypeStruct((vocab // 4, 128), jnp.float32),
        scratch_types=[
            pltpu.VMEM((2, 32, 128), jnp.float32),
            pltpu.VMEM((32, 128), jnp.float32),
            pltpu.VMEM((32, 128), jnp.float32),
            pltpu.SemaphoreType.DMA,
            pltpu.SemaphoreType.DMA,
            pltpu.SemaphoreType.DMA,
            pltpu.SemaphoreType.DMA,
        ],
        compiler_params=pltpu.CompilerParams(use_tc_tiling_on_sc=True,
                                             needs_layout_passes=False),
    )
    def table_transpose(table_t, tl, in_v, ov0, ov1, is0, is1, os0, os1):
        w = _worker_id()
        start = w * per_w
        isems = (is0, is1)
        osems = (os0, os1)
        out_bufs = (ov0, ov1)
        iota = lax.iota(jnp.int32, 16)
        q128 = iota >> 2
        r32 = (iota & 3) * 32

        def in_copy(i, b, sem):
            c = start + i
            return pltpu.make_async_copy(
                table_t.at[:, pl.ds(c * 128, 128)], in_v.at[b], sem)

        def out_copy(i, b, sem):
            c = start + i
            return pltpu.make_async_copy(
                out_bufs[b], tl.at[pl.ds(c * 32, 32)], sem)

        def transpose_block(in_ref, out_ref):
            @plsc.parallel_loop(0, 32, 1, unroll=2)
            def _(h0):
                hd = (h0 + iota) & 31
                cd = r32 + hd
                for jg in range(8):
                    vec = plsc.load_gather(in_ref, [hd, jg * 16 + iota])
                    plsc.store_scatter(out_ref, [jg * 4 + q128, cd], vec)

        in_copy(0, 0, is0).start()

        def pair_body(p, carry):
            for b in (0, 1):
                i = 2 * p + b

                @pl.when(i < per_w - 1)
                def _():
                    in_copy(i + 1, 1 - b, isems[1 - b]).start()

                in_copy(i, b, isems[b]).wait()

                @pl.when(i >= 2)
                def _():
                    out_copy(i - 2, b, osems[b]).wait()

                transpose_block(in_v.at[b], out_bufs[b])
                out_copy(i, b, osems[b]).start()
            return carry

        lax.fori_loop(0, per_w // 2, pair_body, 0)
        out_copy(per_w - 2, 0, os0).wait()
        out_copy(per_w - 1, 1, os1).wait()

        @pl.when(w < n_extra)
        def _():
            c = per_w * _NW + w
            pltpu.sync_copy(table_t.at[:, pl.ds(c * 128, 128)], in_v.at[0])
            transpose_block(in_v.at[0], ov0)
            pltpu.sync_copy(ov0, tl.at[pl.ds(c * 32, 32)])


    return table_transpose


@functools.lru_cache(maxsize=None)
def _make_gather(batch: int, seq: int, vocab: int, hidden: int):
    assert hidden == 32 and batch % 2048 == 0 and seq % 8 == 0
    t_per_w = seq // 8
    n_chunks = t_per_w * 2
    assert n_chunks % 2 == 0
    mesh = plsc.VectorSubcoreMesh(core_axis_name="c", subcore_axis_name="s")

    @functools.partial(
        pl.kernel,
        mesh=mesh,
        out_type=jax.ShapeDtypeStruct((seq, 4, (batch // 128) * 1024),
                                      jnp.float32),
        scratch_types=[
            pltpu.VMEM((2, 512), jnp.int32),
            pltpu.VMEM((2, 512, 32), jnp.float32),
            pltpu.VMEM((4, 4096), jnp.float32),
            pltpu.SemaphoreType.DMA,
            pltpu.SemaphoreType.DMA,
            pltpu.SemaphoreType.DMA,
        ],
        compiler_params=pltpu.CompilerParams(use_tc_tiling_on_sc=False,
                                             needs_layout_passes=False),
    )
    def gather_kernel(ids_t, table_lin, out, idx_v, rows_v, out_v,
                      gs0, gs1, osem):
        w = _worker_id()
        tb = w >> 2
        bb = w & 3
        gsems = (gs0, gs1)
        iota = lax.iota(jnp.int32, 16)

        def t_of(c):
            return tb * t_per_w + (c >> 1)

        def idx_load(c, b):
            b0 = bb * 1024 + (c & 1) * 512
            pltpu.sync_copy(ids_t.at[t_of(c), pl.ds(b0, 512)], idx_v.at[b])

        def gather(b, sem):
            return pltpu.make_async_copy(
                table_lin.at[idx_v.at[b]], rows_v.at[b], sem)

        def out_dma(c, sem):
            wb = bb * 8 + (c & 1) * 4
            return pltpu.make_async_copy(
                out_v, out.at[t_of(c), :, pl.ds(wb * 1024, 4096)], sem)

        def transpose_chunk(b):
            rows = rows_v.at[b]

            @plsc.parallel_loop(0, 32, 1, unroll=2)
            def _(h0):
                hd = (h0 + iota) & 31
                hbv = hd >> 3
                pbase = (hd & 7) * 128 + iota
                for wl in range(4):
                    for l0 in range(8):
                        rv = wl * 128 + l0 * 16 + iota
                        vec = plsc.load_gather(rows, [rv, hd])
                        plsc.store_scatter(
                            out_v, [hbv, pbase + (wl * 1024 + l0 * 16)], vec)

        idx_load(0, 0)
        gather(0, gs0).start()

        def pair_body(p, carry):
            for b in (0, 1):
                c = 2 * p + b

                @pl.when(c < n_chunks - 1)
                def _():
                    idx_load(c + 1, 1 - b)
                    gather(1 - b, gsems[1 - b]).start()

                gather(b, gsems[b]).wait()

                @pl.when(c >= 1)
                def _():
                    out_dma(c - 1, osem).wait()

                transpose_chunk(b)
                out_dma(c, osem).start()
            return carry

        lax.fori_loop(0, n_chunks // 2, pair_body, 0)
        out_dma(n_chunks - 1, osem).wait()

    return gather_kernel


def kernel(input_ids, embedding_weight):
    b, t = input_ids.shape
    vocab, hidden = embedding_weight.shape
    table_t = embedding_weight.T
    tl = _make_table_transpose(vocab, hidden)(table_t)
    n_main = (vocab // 128) * 128
    if n_main < vocab:
        tail = embedding_weight[n_main:].reshape(-1, 128)
        tl = lax.dynamic_update_slice(tl, tail, (n_main // 4, 0))
    table_lin = tl.reshape(vocab, hidden)
    ids_t = input_ids.T.astype(jnp.int32)
    out_lin = _make_gather(b, t, vocab, hidden)(ids_t, table_lin)
    return (out_lin.reshape(t, 4, b // 128, 8, 128)
            .transpose(2, 4, 0, 1, 3).reshape(b, t, hidden))


# speedup vs baseline: 2.7338x; 1.1137x over previous
import functools

import jax
import jax.numpy as jnp
from jax import lax
from jax.experimental import pallas as pl
from jax.experimental.pallas import tpu as pltpu, tpu_sc as plsc

_info = plsc.get_sparse_core_info()
_NC, _NS = _info.num_cores, _info.num_subcores
_NW = _NC * _NS


def _worker_id():
    return lax.axis_index("s") * _NC + lax.axis_index("c")


@functools.lru_cache(maxsize=None)
def _make_table_transpose(vocab: int, hidden: int):
    assert hidden == 32 and vocab % 4 == 0
    n_full = vocab // 128
    per_w = n_full // _NW
    n_extra = n_full - per_w * _NW
    assert per_w % 2 == 0
    mesh = plsc.VectorSubcoreMesh(core_axis_name="c", subcore_axis_name="s")

    @functools.partial(
        pl.kernel,
        mesh=mesh,
        out_type=jax.ShapeDtypeStruct((vocab // 4, 128), jnp.float32),
        scratch_types=[
            pltpu.VMEM((2, 32, 128), jnp.float32),
            pltpu.VMEM((32, 128), jnp.float32),
            pltpu.VMEM((32, 128), jnp.float32),
            pltpu.SemaphoreType.DMA,
            pltpu.SemaphoreType.DMA,
            pltpu.SemaphoreType.DMA,
            pltpu.SemaphoreType.DMA,
        ],
        compiler_params=pltpu.CompilerParams(use_tc_tiling_on_sc=True,
                                             needs_layout_passes=False),
    )
    def table_transpose(table_t, tl, in_v, ov0, ov1, is0, is1, os0, os1):
        w = _worker_id()
        start = w * per_w
        isems = (is0, is1)
        osems = (os0, os1)
        out_bufs = (ov0, ov1)
        iota = lax.iota(jnp.int32, 16)
        q128 = iota >> 2
        r32 = (iota & 3) * 32

        def in_copy(i, b, sem):
            c = start + i
            return pltpu.make_async_copy(
                table_t.at[:, pl.ds(c * 128, 128)], in_v.at[b], sem)

        def out_copy(i, b, sem):
            c = start + i
            return pltpu.make_async_copy(
                out_bufs[b], tl.at[pl.ds(c * 32, 32)], sem)

        def transpose_block(in_ref, out_ref):
            @plsc.parallel_loop(0, 32, 1, unroll=2)
            def _(h0):
                hd = (h0 + iota) & 31
                cd = r32 + hd
                for jg in range(8):
                    vec = plsc.load_gather(in_ref, [hd, jg * 16 + iota])
                    plsc.store_scatter(out_ref, [jg * 4 + q128, cd], vec)

        in_copy(0, 0, is0).start()

        def pair_body(p, carry):
            for b in (0, 1):
                i = 2 * p + b

                @pl.when(i < per_w - 1)
                def _():
                    in_copy(i + 1, 1 - b, isems[1 - b]).start()

                in_copy(i, b, isems[b]).wait()

                @pl.when(i >= 2)
                def _():
                    out_copy(i - 2, b, osems[b]).wait()

                transpose_block(in_v.at[b], out_bufs[b])
                out_copy(i, b, osems[b]).start()
            return carry

        lax.fori_loop(0, per_w // 2, pair_body, 0)
        out_copy(per_w - 2, 0, os0).wait()
        out_copy(per_w - 1, 1, os1).wait()

        @pl.when(w < n_extra)
        def _():
            c = per_w * _NW + w
            pltpu.sync_copy(table_t.at[:, pl.ds(c * 128, 128)], in_v.at[0])
            transpose_block(in_v.at[0], ov0)
            pltpu.sync_copy(ov0, tl.at[pl.ds(c * 32, 32)])


    return table_transpose


@functools.lru_cache(maxsize=None)
def _make_gather(batch: int, seq: int, vocab: int, hidden: int):
    assert hidden == 32 and batch == 4096 and seq % 8 == 0
    t_per_w = seq // 8
    n_chunks = t_per_w * 2
    assert n_chunks % 2 == 0
    mesh = plsc.VectorSubcoreMesh(core_axis_name="c", subcore_axis_name="s")

    @functools.partial(
        pl.kernel,
        mesh=mesh,
        out_type=jax.ShapeDtypeStruct((seq, 4, (batch // 128) * 1024),
                                      jnp.float32),
        scratch_types=[
            pltpu.VMEM((2, 512), jnp.int32),
            pltpu.VMEM((2, 512, 32), jnp.float32),
            pltpu.VMEM((4, 4096), jnp.float32),
            pltpu.SemaphoreType.DMA,
            pltpu.SemaphoreType.DMA,
            pltpu.SemaphoreType.DMA,
        ],
        compiler_params=pltpu.CompilerParams(use_tc_tiling_on_sc=False,
                                             needs_layout_passes=False),
    )
    def gather_kernel(ids_t, table_lin, out, idx_v, rows_v, out_v,
                      gs0, gs1, osem):
        w = _worker_id()
        tb = w >> 2
        bb = w & 3
        gsems = (gs0, gs1)
        iota = lax.iota(jnp.int32, 16)

        def t_of(c):
            return tb * t_per_w + (c >> 1)

        def idx_load(c, b):
            b0 = bb * 1024 + (c & 1) * 512
            pltpu.sync_copy(ids_t.at[t_of(c), pl.ds(b0, 512)], idx_v.at[b])

        def gather(b, sem):
            return pltpu.make_async_copy(
                table_lin.at[idx_v.at[b]], rows_v.at[b], sem)

        def out_dma(c, sem):
            wb = bb * 8 + (c & 1) * 4
            return pltpu.make_async_copy(
                out_v, out.at[t_of(c), :, pl.ds(wb * 1024, 4096)], sem)

        def transpose_chunk(b):
            rows = rows_v.at[b]

            @plsc.parallel_loop(0, 32, 1, unroll=4)
            def _(h0):
                hd = (h0 + iota) & 31
                hbv = hd >> 3
                pbase = (hd & 7) * 128 + iota
                for wl in range(4):
                    for l0 in range(8):
                        rv = wl * 128 + l0 * 16 + iota
                        vec = plsc.load_gather(rows, [rv, hd])
                        plsc.store_scatter(
                            out_v, [hbv, pbase + (wl * 1024 + l0 * 16)], vec)

        idx_load(0, 0)
        gather(0, gs0).start()

        def pair_body(p, carry):
            for b in (0, 1):
                c = 2 * p + b

                @pl.when(c < n_chunks - 1)
                def _():
                    idx_load(c + 1, 1 - b)
                    gather(1 - b, gsems[1 - b]).start()

                gather(b, gsems[b]).wait()

                @pl.when(c >= 1)
                def _():
                    out_dma(c - 1, osem).wait()

                transpose_chunk(b)
                out_dma(c, osem).start()
            return carry

        lax.fori_loop(0, n_chunks // 2, pair_body, 0)
        out_dma(n_chunks - 1, osem).wait()

    return gather_kernel


def kernel(input_ids, embedding_weight):
    b, t = input_ids.shape
    vocab, hidden = embedding_weight.shape
    table_t = embedding_weight.T
    tl = _make_table_transpose(vocab, hidden)(table_t)
    n_main = (vocab // 128) * 128
    if n_main < vocab:
        tail = embedding_weight[n_main:].reshape(-1, 128)
        tl = lax.dynamic_update_slice(tl, tail, (n_main // 4, 0))
    table_lin = tl.reshape(vocab, hidden)
    ids_t = input_ids.T.astype(jnp.int32)
    out_lin = _make_gather(b, t, vocab, hidden)(ids_t, table_lin)
    return (out_lin.reshape(t, 4, b // 128, 8, 128)
            .transpose(2, 4, 0, 1, 3).reshape(b, t, hidden))


# speedup vs baseline: 2.7494x; 1.0057x over previous
import functools

import jax
import jax.numpy as jnp
from jax import lax
from jax.experimental import pallas as pl
from jax.experimental.pallas import tpu as pltpu, tpu_sc as plsc

_info = plsc.get_sparse_core_info()
_NC, _NS = _info.num_cores, _info.num_subcores
_NW = _NC * _NS


def _worker_id():
    return lax.axis_index("s") * _NC + lax.axis_index("c")


@functools.lru_cache(maxsize=None)
def _make_table_transpose(vocab: int, hidden: int):
    assert hidden == 32 and vocab % 4 == 0
    n_full = vocab // 128
    per_w = n_full // _NW
    n_extra = n_full - per_w * _NW
    assert per_w % 2 == 0
    mesh = plsc.VectorSubcoreMesh(core_axis_name="c", subcore_axis_name="s")

    @functools.partial(
        pl.kernel,
        mesh=mesh,
        out_type=jax.ShapeDtypeStruct((vocab // 4, 128), jnp.float32),
        scratch_types=[
            pltpu.VMEM((2, 32, 128), jnp.float32),
            pltpu.VMEM((32, 128), jnp.float32),
            pltpu.VMEM((32, 128), jnp.float32),
            pltpu.SemaphoreType.DMA,
            pltpu.SemaphoreType.DMA,
            pltpu.SemaphoreType.DMA,
            pltpu.SemaphoreType.DMA,
        ],
        compiler_params=pltpu.CompilerParams(use_tc_tiling_on_sc=True,
                                             needs_layout_passes=False),
    )
    def table_transpose(table_t, tl, in_v, ov0, ov1, is0, is1, os0, os1):
        w = _worker_id()
        start = w * per_w
        isems = (is0, is1)
        osems = (os0, os1)
        out_bufs = (ov0, ov1)
        iota = lax.iota(jnp.int32, 16)
        q128 = iota >> 2
        r32 = (iota & 3) * 32

        def in_copy(i, b, sem):
            c = start + i
            return pltpu.make_async_copy(
                table_t.at[:, pl.ds(c * 128, 128)], in_v.at[b], sem)

        def out_copy(i, b, sem):
            c = start + i
            return pltpu.make_async_copy(
                out_bufs[b], tl.at[pl.ds(c * 32, 32)], sem)

        def transpose_block(in_ref, out_ref):
            @plsc.parallel_loop(0, 32, 1, unroll=4)
            def _(h0):
                hd = (h0 + iota) & 31
                cd = r32 + hd
                for jg in range(8):
                    vec = plsc.load_gather(in_ref, [hd, jg * 16 + iota])
                    plsc.store_scatter(out_ref, [jg * 4 + q128, cd], vec)

        in_copy(0, 0, is0).start()

        def pair_body(p, carry):
            for b in (0, 1):
                i = 2 * p + b

                @pl.when(i < per_w - 1)
                def _():
                    in_copy(i + 1, 1 - b, isems[1 - b]).start()

                in_copy(i, b, isems[b]).wait()

                @pl.when(i >= 2)
                def _():
                    out_copy(i - 2, b, osems[b]).wait()

                transpose_block(in_v.at[b], out_bufs[b])
                out_copy(i, b, osems[b]).start()
            return carry

        lax.fori_loop(0, per_w // 2, pair_body, 0)
        out_copy(per_w - 2, 0, os0).wait()
        out_copy(per_w - 1, 1, os1).wait()

        @pl.when(w < n_extra)
        def _():
            c = per_w * _NW + w
            pltpu.sync_copy(table_t.at[:, pl.ds(c * 128, 128)], in_v.at[0])
            transpose_block(in_v.at[0], ov0)
            pltpu.sync_copy(ov0, tl.at[pl.ds(c * 32, 32)])


    return table_transpose


@functools.lru_cache(maxsize=None)
def _make_gather(batch: int, seq: int, vocab: int, hidden: int):
    assert hidden == 32 and batch == 4096 and seq % 8 == 0
    t_per_w = seq // 8
    n_chunks = t_per_w * 2
    assert n_chunks % 2 == 0
    mesh = plsc.VectorSubcoreMesh(core_axis_name="c", subcore_axis_name="s")

    @functools.partial(
        pl.kernel,
        mesh=mesh,
        out_type=jax.ShapeDtypeStruct((seq, 4, (batch // 128) * 1024),
                                      jnp.float32),
        scratch_types=[
            pltpu.VMEM((2, 512), jnp.int32),
            pltpu.VMEM((2, 512, 32), jnp.float32),
            pltpu.VMEM((4, 4096), jnp.float32),
            pltpu.SemaphoreType.DMA,
            pltpu.SemaphoreType.DMA,
            pltpu.SemaphoreType.DMA,
        ],
        compiler_params=pltpu.CompilerParams(use_tc_tiling_on_sc=False,
                                             needs_layout_passes=False),
    )
    def gather_kernel(ids_t, table_lin, out, idx_v, rows_v, out_v,
                      gs0, gs1, osem):
        w = _worker_id()
        tb = w >> 2
        bb = w & 3
        gsems = (gs0, gs1)
        iota = lax.iota(jnp.int32, 16)

        def t_of(c):
            return tb * t_per_w + (c >> 1)

        def idx_load(c, b):
            b0 = bb * 1024 + (c & 1) * 512
            pltpu.sync_copy(ids_t.at[t_of(c), pl.ds(b0, 512)], idx_v.at[b])

        def gather(b, sem):
            return pltpu.make_async_copy(
                table_lin.at[idx_v.at[b]], rows_v.at[b], sem)

        def out_dma(c, sem):
            wb = bb * 8 + (c & 1) * 4
            return pltpu.make_async_copy(
                out_v, out.at[t_of(c), :, pl.ds(wb * 1024, 4096)], sem)

        def transpose_chunk(b):
            rows = rows_v.at[b]

            @plsc.parallel_loop(0, 32, 1, unroll=4)
            def _(h0):
                hd = (h0 + iota) & 31
                hbv = hd >> 3
                pbase = (hd & 7) * 128 + iota
                for wl in range(4):
                    for l0 in range(8):
                        rv = wl * 128 + l0 * 16 + iota
                        vec = plsc.load_gather(rows, [rv, hd])
                        plsc.store_scatter(
                            out_v, [hbv, pbase + (wl * 1024 + l0 * 16)], vec)

        idx_load(0, 0)
        gather(0, gs0).start()

        def pair_body(p, carry):
            for b in (0, 1):
                c = 2 * p + b

                @pl.when(c < n_chunks - 1)
                def _():
                    idx_load(c + 1, 1 - b)
                    gather(1 - b, gsems[1 - b]).start()

                gather(b, gsems[b]).wait()

                @pl.when(c >= 1)
                def _():
                    out_dma(c - 1, osem).wait()

                transpose_chunk(b)
                out_dma(c, osem).start()
            return carry

        lax.fori_loop(0, n_chunks // 2, pair_body, 0)
        out_dma(n_chunks - 1, osem).wait()

    return gather_kernel


def kernel(input_ids, embedding_weight):
    b, t = input_ids.shape
    vocab, hidden = embedding_weight.shape
    table_t = embedding_weight.T
    tl = _make_table_transpose(vocab, hidden)(table_t)
    n_main = (vocab // 128) * 128
    if n_main < vocab:
        tail = embedding_weight[n_main:].reshape(-1, 128)
        tl = lax.dynamic_update_slice(tl, tail, (n_main // 4, 0))
    table_lin = tl.reshape(vocab, hidden)
    ids_t = input_ids.T.astype(jnp.int32)
    out_lin = _make_gather(b, t, vocab, hidden)(ids_t, table_lin)
    return (out_lin.reshape(t, 4, b // 128, 8, 128)
            .transpose(2, 4, 0, 1, 3).reshape(b, t, hidden))


# speedup vs baseline: 2.9270x; 1.0646x over previous
import functools

import jax
import jax.numpy as jnp
from jax import lax
from jax.experimental import pallas as pl
from jax.experimental.pallas import tpu as pltpu, tpu_sc as plsc

_info = plsc.get_sparse_core_info()
_NC, _NS = _info.num_cores, _info.num_subcores
_NW = _NC * _NS


def _worker_id():
    return lax.axis_index("s") * _NC + lax.axis_index("c")


@functools.lru_cache(maxsize=None)
def _make_table_transpose(vocab: int, hidden: int):
    assert hidden == 32 and vocab % 4 == 0
    n_full = vocab // 128
    per_w = n_full // _NW
    n_extra = n_full - per_w * _NW
    assert per_w % 2 == 0
    mesh = plsc.VectorSubcoreMesh(core_axis_name="c", subcore_axis_name="s")

    @functools.partial(
        pl.kernel,
        mesh=mesh,
        out_type=jax.ShapeDtypeStruct((vocab // 4, 128), jnp.float32),
        scratch_types=[
            pltpu.VMEM((2, 32, 128), jnp.float32),
            pltpu.VMEM((32, 128), jnp.float32),
            pltpu.VMEM((32, 128), jnp.float32),
            pltpu.SemaphoreType.DMA,
            pltpu.SemaphoreType.DMA,
            pltpu.SemaphoreType.DMA,
            pltpu.SemaphoreType.DMA,
        ],
        compiler_params=pltpu.CompilerParams(use_tc_tiling_on_sc=True,
                                             needs_layout_passes=False),
    )
    def table_transpose(table_t, tl, in_v, ov0, ov1, is0, is1, os0, os1):
        w = _worker_id()
        start = w * per_w
        isems = (is0, is1)
        osems = (os0, os1)
        out_bufs = (ov0, ov1)
        iota = lax.iota(jnp.int32, 16)
        q128 = iota >> 2
        r32 = (iota & 3) * 32

        def in_copy(i, b, sem):
            c = start + i
            return pltpu.make_async_copy(
                table_t.at[:, pl.ds(c * 128, 128)], in_v.at[b], sem)

        def out_copy(i, b, sem):
            c = start + i
            return pltpu.make_async_copy(
                out_bufs[b], tl.at[pl.ds(c * 32, 32)], sem)

        def transpose_block(in_ref, out_ref):
            @plsc.parallel_loop(0, 32, 1, unroll=4)
            def _(h0):
                hd = (h0 + iota) & 31
                cd = r32 + hd
                for jg in range(8):
                    vec = plsc.load_gather(in_ref, [hd, jg * 16 + iota])
                    plsc.store_scatter(out_ref, [jg * 4 + q128, cd], vec)

        in_copy(0, 0, is0).start()

        def pair_body(p, carry):
            for b in (0, 1):
                i = 2 * p + b

                @pl.when(i < per_w - 1)
                def _():
                    in_copy(i + 1, 1 - b, isems[1 - b]).start()

                in_copy(i, b, isems[b]).wait()

                @pl.when(i >= 2)
                def _():
                    out_copy(i - 2, b, osems[b]).wait()

                transpose_block(in_v.at[b], out_bufs[b])
                out_copy(i, b, osems[b]).start()
            return carry

        lax.fori_loop(0, per_w // 2, pair_body, 0)
        out_copy(per_w - 2, 0, os0).wait()
        out_copy(per_w - 1, 1, os1).wait()

        @pl.when(w < n_extra)
        def _():
            c = per_w * _NW + w
            pltpu.sync_copy(table_t.at[:, pl.ds(c * 128, 128)], in_v.at[0])
            transpose_block(in_v.at[0], ov0)
            pltpu.sync_copy(ov0, tl.at[pl.ds(c * 32, 32)])


    return table_transpose


@functools.lru_cache(maxsize=None)
def _make_gather(batch: int, seq: int, vocab: int, hidden: int):
    assert hidden == 32 and batch == 4096 and seq % 8 == 0
    t_per_w = seq // 8
    n_chunks = t_per_w
    assert n_chunks % 2 == 1
    mesh = plsc.VectorSubcoreMesh(core_axis_name="c", subcore_axis_name="s")

    @functools.partial(
        pl.kernel,
        mesh=mesh,
        out_type=jax.ShapeDtypeStruct((seq, 4, (batch // 128) * 1024),
                                      jnp.float32),
        scratch_types=[
            pltpu.VMEM((2, 1024), jnp.int32),
            pltpu.VMEM((2, 1024, 32), jnp.float32),
            pltpu.VMEM((4, 8192), jnp.float32),
            pltpu.SemaphoreType.DMA,
            pltpu.SemaphoreType.DMA,
            pltpu.SemaphoreType.DMA,
        ],
        compiler_params=pltpu.CompilerParams(use_tc_tiling_on_sc=False,
                                             needs_layout_passes=False),
    )
    def gather_kernel(ids_t, table_lin, out, idx_v, rows_v, out_v,
                      gs0, gs1, osem):
        w = _worker_id()
        tb = w >> 2
        bb = w & 3
        gsems = (gs0, gs1)
        iota = lax.iota(jnp.int32, 16)

        def t_of(c):
            return tb * t_per_w + c

        def idx_load(c, b):
            pltpu.sync_copy(ids_t.at[t_of(c), pl.ds(bb * 1024, 1024)],
                            idx_v.at[b])

        def gather(b, sem):
            return pltpu.make_async_copy(
                table_lin.at[idx_v.at[b]], rows_v.at[b], sem)

        def out_dma(c, sem):
            return pltpu.make_async_copy(
                out_v, out.at[t_of(c), :, pl.ds(bb * 8192, 8192)], sem)

        def transpose_chunk(b):
            rows = rows_v.at[b]

            @plsc.parallel_loop(0, 32, 1, unroll=4)
            def _(h0):
                hd = (h0 + iota) & 31
                hbv = hd >> 3
                pbase = (hd & 7) * 128 + iota
                for wl in range(8):
                    for l0 in range(8):
                        rv = wl * 128 + l0 * 16 + iota
                        vec = plsc.load_gather(rows, [rv, hd])
                        plsc.store_scatter(
                            out_v, [hbv, pbase + (wl * 1024 + l0 * 16)], vec)

        idx_load(0, 0)
        gather(0, gs0).start()

        def step(c, s):
            @pl.when(c < n_chunks - 1)
            def _():
                idx_load(c + 1, 1 - s)
                gather(1 - s, gsems[1 - s]).start()

            gather(s, gsems[s]).wait()

            @pl.when(c >= 1)
            def _():
                out_dma(c - 1, osem).wait()

            transpose_chunk(s)
            out_dma(c, osem).start()

        step(0, 0)

        def pair_body(p, carry):
            step(2 * p + 1, 1)
            step(2 * p + 2, 0)
            return carry

        lax.fori_loop(0, (n_chunks - 1) // 2, pair_body, 0)
        out_dma(n_chunks - 1, osem).wait()

    return gather_kernel


def kernel(input_ids, embedding_weight):
    b, t = input_ids.shape
    vocab, hidden = embedding_weight.shape
    table_t = embedding_weight.T
    tl = _make_table_transpose(vocab, hidden)(table_t)
    n_main = (vocab // 128) * 128
    if n_main < vocab:
        tail = embedding_weight[n_main:].reshape(-1, 128)
        tl = lax.dynamic_update_slice(tl, tail, (n_main // 4, 0))
    table_lin = tl.reshape(vocab, hidden)
    ids_t = input_ids.T.astype(jnp.int32)
    out_lin = _make_gather(b, t, vocab, hidden)(ids_t, table_lin)
    return (out_lin.reshape(t, 4, b // 128, 8, 128)
            .transpose(2, 4, 0, 1, 3).reshape(b, t, hidden))


# speedup vs baseline: 3.5825x; 1.2240x over previous
import functools

import jax
import jax.numpy as jnp
from jax import lax
from jax.experimental import pallas as pl
from jax.experimental.pallas import tpu as pltpu, tpu_sc as plsc

_info = plsc.get_sparse_core_info()
_NC, _NS = _info.num_cores, _info.num_subcores
_NW = _NC * _NS


def _worker_id():
    return lax.axis_index("s") * _NC + lax.axis_index("c")


@functools.lru_cache(maxsize=None)
def _make_table_transpose(vocab: int, hidden: int):
    assert hidden == 32 and vocab % 4 == 0
    n_full = vocab // 128
    assert n_full % 4 == 0
    n_quad = n_full // 4
    per_w = n_quad // _NW
    n_extra = n_quad - per_w * _NW
    mesh = plsc.VectorSubcoreMesh(core_axis_name="c", subcore_axis_name="s")

    @functools.partial(
        pl.kernel,
        mesh=mesh,
        out_type=jax.ShapeDtypeStruct((vocab // 4, 128), jnp.float32),
        scratch_types=[
            pltpu.VMEM((2, 32, 512), jnp.float32),
            pltpu.VMEM((128, 128), jnp.float32),
            pltpu.VMEM((128, 128), jnp.float32),
            pltpu.SemaphoreType.DMA,
            pltpu.SemaphoreType.DMA,
            pltpu.SemaphoreType.DMA,
            pltpu.SemaphoreType.DMA,
        ],
        compiler_params=pltpu.CompilerParams(use_tc_tiling_on_sc=True,
                                             needs_layout_passes=False),
    )
    def table_transpose(table_t, tl, in_v, ov0, ov1, is0, is1, os0, os1):
        w = _worker_id()
        start = w * per_w
        isems = (is0, is1)
        osems = (os0, os1)
        out_bufs = (ov0, ov1)
        iota = lax.iota(jnp.int32, 16)
        q128 = iota >> 2
        r32 = (iota & 3) * 32

        def in_copy(i, b, sem):
            c = start + i
            return pltpu.make_async_copy(
                table_t.at[:, pl.ds(c * 512, 512)], in_v.at[b], sem)

        def out_copy(i, b, sem):
            c = start + i
            return pltpu.make_async_copy(
                out_bufs[b], tl.at[pl.ds(c * 128, 128)], sem)

        def transpose_block(in_ref, out_ref):
            @plsc.parallel_loop(0, 32, 1, unroll=4)
            def _(h0):
                hd = (h0 + iota) & 31
                cd = r32 + hd
                for sub in range(4):
                    for jg in range(8):
                        vec = plsc.load_gather(
                            in_ref, [hd, sub * 128 + jg * 16 + iota])
                        plsc.store_scatter(
                            out_ref, [sub * 32 + jg * 4 + q128, cd], vec)

        in_copy(0, 0, is0).start()

        def step(i, b):
            @pl.when(i < per_w - 1)
            def _():
                in_copy(i + 1, 1 - b, isems[1 - b]).start()

            in_copy(i, b, isems[b]).wait()

            @pl.when(i >= 2)
            def _():
                out_copy(i - 2, b, osems[b]).wait()

            transpose_block(in_v.at[b], out_bufs[b])
            out_copy(i, b, osems[b]).start()

        step(0, 0)

        def pair_body(p, carry):
            step(2 * p + 1, 1)
            step(2 * p + 2, 0)
            return carry

        lax.fori_loop(0, (per_w - 1) // 2, pair_body, 0)
        out_copy(per_w - 2, (per_w - 2) & 1, osems[(per_w - 2) & 1]).wait()
        out_copy(per_w - 1, (per_w - 1) & 1, osems[(per_w - 1) & 1]).wait()

        @pl.when(w < n_extra)
        def _():
            c = per_w * _NW + w
            pltpu.sync_copy(table_t.at[:, pl.ds(c * 512, 512)], in_v.at[0])
            transpose_block(in_v.at[0], ov0)
            pltpu.sync_copy(ov0, tl.at[pl.ds(c * 128, 128)])


    return table_transpose


@functools.lru_cache(maxsize=None)
def _make_gather(batch: int, seq: int, vocab: int, hidden: int):
    assert hidden == 32 and batch == 4096 and seq % 8 == 0
    t_per_w = seq // 8
    n_chunks = t_per_w
    assert n_chunks % 2 == 1
    mesh = plsc.VectorSubcoreMesh(core_axis_name="c", subcore_axis_name="s")

    @functools.partial(
        pl.kernel,
        mesh=mesh,
        out_type=jax.ShapeDtypeStruct((seq, 4, (batch // 128) * 1024),
                                      jnp.float32),
        scratch_types=[
            pltpu.VMEM((2, 1024), jnp.int32),
            pltpu.VMEM((2, 1024, 32), jnp.float32),
            pltpu.VMEM((4, 8192), jnp.float32),
            pltpu.SemaphoreType.DMA,
            pltpu.SemaphoreType.DMA,
            pltpu.SemaphoreType.DMA,
        ],
        compiler_params=pltpu.CompilerParams(use_tc_tiling_on_sc=False,
                                             needs_layout_passes=False),
    )
    def gather_kernel(ids_t, table_lin, out, idx_v, rows_v, out_v,
                      gs0, gs1, osem):
        w = _worker_id()
        tb = w >> 2
        bb = w & 3
        gsems = (gs0, gs1)
        iota = lax.iota(jnp.int32, 16)

        def t_of(c):
            return tb * t_per_w + c

        def idx_load(c, b):
            pltpu.sync_copy(ids_t.at[t_of(c), pl.ds(bb * 1024, 1024)],
                            idx_v.at[b])

        def gather(b, sem):
            return pltpu.make_async_copy(
                table_lin.at[idx_v.at[b]], rows_v.at[b], sem)

        def out_dma(c, sem):
            return pltpu.make_async_copy(
                out_v, out.at[t_of(c), :, pl.ds(bb * 8192, 8192)], sem)

        def transpose_chunk(b):
            rows = rows_v.at[b]

            @plsc.parallel_loop(0, 32, 1, unroll=4)
            def _(h0):
                hd = (h0 + iota) & 31
                hbv = hd >> 3
                pbase = (hd & 7) * 128 + iota
                for wl in range(8):
                    for l0 in range(8):
                        rv = wl * 128 + l0 * 16 + iota
                        vec = plsc.load_gather(rows, [rv, hd])
                        plsc.store_scatter(
                            out_v, [hbv, pbase + (wl * 1024 + l0 * 16)], vec)

        idx_load(0, 0)
        gather(0, gs0).start()

        def step(c, s):
            @pl.when(c < n_chunks - 1)
            def _():
                idx_load(c + 1, 1 - s)
                gather(1 - s, gsems[1 - s]).start()

            gather(s, gsems[s]).wait()

            @pl.when(c >= 1)
            def _():
                out_dma(c - 1, osem).wait()

            transpose_chunk(s)
            out_dma(c, osem).start()

        step(0, 0)

        def pair_body(p, carry):
            step(2 * p + 1, 1)
            step(2 * p + 2, 0)
            return carry

        lax.fori_loop(0, (n_chunks - 1) // 2, pair_body, 0)
        out_dma(n_chunks - 1, osem).wait()

    return gather_kernel


def kernel(input_ids, embedding_weight):
    b, t = input_ids.shape
    vocab, hidden = embedding_weight.shape
    table_t = embedding_weight.T
    tl = _make_table_transpose(vocab, hidden)(table_t)
    n_main = (vocab // 128) * 128
    if n_main < vocab:
        tail = embedding_weight[n_main:].reshape(-1, 128)
        tl = lax.dynamic_update_slice(tl, tail, (n_main // 4, 0))
    table_lin = tl.reshape(vocab, hidden)
    ids_t = input_ids.T.astype(jnp.int32)
    out_lin = _make_gather(b, t, vocab, hidden)(ids_t, table_lin)
    return (out_lin.reshape(t, 4, b // 128, 8, 128)
            .transpose(2, 4, 0, 1, 3).reshape(b, t, hidden))
